# baseline (speedup 1.0000x reference)
import jax
import jax.numpy as jnp
from jax.experimental import pallas as pl
from jax.experimental.pallas import tpu as pltpu

_TEMP = 0.07
_GROUPS = ((0, 2), (2, 17), (17, 53), (53, 89), (89, 116))
_NPROTO = 116
_PPAD = 128
_BB = 256


def _body(f_ref, w_ref, b_ref, p1_ref, p2_ref, p3_ref, p4_ref, p5_ref,
          logits_ref, ids_ref, pn_ref):
    sem = w_ref.shape[1]

    @pl.when(pl.program_id(0) == 0)
    def _prep_protos():
        p = jnp.concatenate(
            [p1_ref[...], p2_ref[...], p3_ref[...], p4_ref[...], p5_ref[...],
             jnp.zeros((_PPAD - _NPROTO, sem), jnp.float32)], axis=0)
        pnorm = jnp.sqrt(jnp.sum(p * p, axis=-1, keepdims=True))
        pn_ref[...] = p / jnp.maximum(pnorm, 1e-12)

    z = jnp.dot(f_ref[...], w_ref[...], preferred_element_type=jnp.float32)
    z = z + b_ref[...].reshape(1, sem)
    znorm = jnp.sqrt(jnp.sum(z * z, axis=-1, keepdims=True))
    zn = z / jnp.maximum(znorm, 1e-12)
    logits = jax.lax.dot_general(
        zn, pn_ref[...], (((1,), (1,)), ((), ())),
        preferred_element_type=jnp.float32,
    ) / _TEMP
    logits_ref[...] = logits[:, :_NPROTO]
    colf = jax.lax.broadcasted_iota(jnp.int32, logits.shape, 1).astype(jnp.float32)
    parts = []
    for s, e in _GROUPS:
        mask = (colf >= s) & (colf < e)
        masked = jnp.where(mask, logits, -jnp.inf)
        m = jnp.max(masked, axis=-1, keepdims=True)
        cand = jnp.where(masked == m, colf, float(_PPAD))
        parts.append(jnp.min(cand, axis=-1, keepdims=True) - s)
    ids_ref[...] = jnp.concatenate(parts, axis=1).astype(jnp.int32)


def kernel(features, proj_w, proj_b, proto_gender, proto_hair, proto_top,
           proto_pants, proto_shoes):
    batch, feat = features.shape
    sem = proj_w.shape[1]
    protos = (proto_gender, proto_hair, proto_top, proto_pants, proto_shoes)

    grid = (batch // _BB,)
    all_logits, ids = pl.pallas_call(
        _body,
        grid=grid,
        in_specs=[
            pl.BlockSpec((_BB, feat), lambda i: (i, 0)),
            pl.BlockSpec((feat, sem), lambda i: (0, 0)),
            pl.BlockSpec((sem,), lambda i: (0,)),
        ] + [
            pl.BlockSpec(p.shape, lambda i: (0, 0)) for p in protos
        ],
        out_specs=[
            pl.BlockSpec((_BB, _NPROTO), lambda i: (i, 0)),
            pl.BlockSpec((_BB, len(_GROUPS)), lambda i: (i, 0)),
        ],
        out_shape=[
            jax.ShapeDtypeStruct((batch, _NPROTO), jnp.float32),
            jax.ShapeDtypeStruct((batch, len(_GROUPS)), jnp.int32),
        ],
        scratch_shapes=[pltpu.VMEM((_PPAD, sem), jnp.float32)],
        compiler_params=pltpu.CompilerParams(
            dimension_semantics=("arbitrary",),
        ),
    )(features, proj_w, proj_b, *protos)
    return all_logits, ids


# speedup vs baseline: 1.1877x; 1.1877x over previous
import jax
import jax.numpy as jnp
from jax.experimental import pallas as pl
from jax.experimental.pallas import tpu as pltpu

_TEMP = 0.07
_GROUPS = ((0, 2), (2, 17), (17, 53), (53, 89), (89, 116))
_NPROTO = 116
_PPAD = 128
_BB = 1024


def _body(f_ref, w_ref, b_ref, p1_ref, p2_ref, p3_ref, p4_ref, p5_ref,
          logits_ref, ids_ref, pn_ref):
    sem = w_ref.shape[1]

    @pl.when(pl.program_id(0) == 0)
    def _prep_protos():
        p = jnp.concatenate(
            [p1_ref[...], p2_ref[...], p3_ref[...], p4_ref[...], p5_ref[...],
             jnp.zeros((_PPAD - _NPROTO, sem), jnp.float32)], axis=0)
        pnorm = jnp.sqrt(jnp.sum(p * p, axis=-1, keepdims=True))
        pn_ref[...] = p / jnp.maximum(pnorm, 1e-12)

    z = jnp.dot(f_ref[...], w_ref[...], preferred_element_type=jnp.float32)
    z = z + b_ref[...].reshape(1, sem)
    znorm = jnp.sqrt(jnp.sum(z * z, axis=-1, keepdims=True))
    zn = z / jnp.maximum(znorm, 1e-12)
    logits = jax.lax.dot_general(
        zn, pn_ref[...], (((1,), (1,)), ((), ())),
        preferred_element_type=jnp.float32,
    ) / _TEMP
    logits_ref[...] = logits[:, :_NPROTO]
    colf = jax.lax.broadcasted_iota(jnp.int32, logits.shape, 1).astype(jnp.float32)
    parts = []
    for s, e in _GROUPS:
        mask = (colf >= s) & (colf < e)
        masked = jnp.where(mask, logits, -jnp.inf)
        m = jnp.max(masked, axis=-1, keepdims=True)
        cand = jnp.where(masked == m, colf, float(_PPAD))
        parts.append(jnp.min(cand, axis=-1, keepdims=True) - s)
    ids_ref[...] = jnp.concatenate(parts, axis=1).astype(jnp.int32)


def kernel(features, proj_w, proj_b, proto_gender, proto_hair, proto_top,
           proto_pants, proto_shoes):
    batch, feat = features.shape
    sem = proj_w.shape[1]
    protos = (proto_gender, proto_hair, proto_top, proto_pants, proto_shoes)

    grid = (batch // _BB,)
    all_logits, ids = pl.pallas_call(
        _body,
        grid=grid,
        in_specs=[
            pl.BlockSpec((_BB, feat), lambda i: (i, 0)),
            pl.BlockSpec((feat, sem), lambda i: (0, 0)),
            pl.BlockSpec((sem,), lambda i: (0,)),
        ] + [
            pl.BlockSpec(p.shape, lambda i: (0, 0)) for p in protos
        ],
        out_specs=[
            pl.BlockSpec((_BB, _NPROTO), lambda i: (i, 0)),
            pl.BlockSpec((_BB, len(_GROUPS)), lambda i: (i, 0)),
        ],
        out_shape=[
            jax.ShapeDtypeStruct((batch, _NPROTO), jnp.float32),
            jax.ShapeDtypeStruct((batch, len(_GROUPS)), jnp.int32),
        ],
        scratch_shapes=[pltpu.VMEM((_PPAD, sem), jnp.float32)],
        compiler_params=pltpu.CompilerParams(
            dimension_semantics=("arbitrary",),
        ),
    )(features, proj_w, proj_b, *protos)
    return all_logits, ids
